# baseline (speedup 1.0000x reference)
import jax
import jax.numpy as jnp
from jax import lax
from jax.experimental import pallas as pl
from jax.experimental.pallas import tpu as pltpu

_EPS = 1e-07
_VOCAB = 30522
_NTOK = 1024
_TPS = 8
_GSTEPS = _NTOK // _TPS


def _gather_body(cblk_ref, *refs):
    in_refs = refs[:_TPS]
    out_ref = refs[_TPS]
    g = pl.program_id(0)
    sub = lax.broadcasted_iota(jnp.int32, (_TPS, 128), 0)
    acc = jnp.zeros((_TPS, 128), jnp.float32)
    for j in range(_TPS):
        acc = jnp.where(sub == j, in_refs[j][0, :, :], acc)
    out_ref[...] = acc.reshape(1, _TPS, 128)
    del g, cblk_ref


def _loss_body(pblk_ref, yt_ref, um_ref, out_ref):
    yt = yt_ref[...]
    um = um_ref[...]
    pblk = pblk_ref[...]
    lane = lax.broadcasted_iota(jnp.int32, (_NTOK, 128), 1)
    p = jnp.sum(
        jnp.where(lane == (yt & 127), pblk, 0.0), axis=1, keepdims=True
    )
    cnt = jnp.sum((yt == um).astype(jnp.float32), axis=1, keepdims=True)
    alpha = jnp.where(
        (yt >= 2) & (cnt > 0.0),
        lax.rsqrt(jnp.maximum(cnt, 1e-20)),
        0.0,
    )
    keep = (yt != 0).astype(jnp.float32)
    a = alpha * keep
    pc = jnp.clip(p, _EPS, 1.0 - _EPS)
    om = 1.0 - pc
    num = jnp.sum(a * om * om * (-jnp.log(pc)))
    den = jnp.sum(a)
    out_ref[...] = (num / den).reshape(1, 1)


def kernel(y_pred, y_true):
    yt = y_true[:, :, 0].reshape(-1)
    um = y_true[:, :, 1].reshape(-1)
    cblk = yt >> 7

    in_specs = [
        pl.BlockSpec(
            (1, _TPS, 128),
            lambda g, cref, j=j: (g // 16, g % 16, cref[_TPS * g + j]),
        )
        for j in range(_TPS)
    ]
    grid_spec = pltpu.PrefetchScalarGridSpec(
        num_scalar_prefetch=1,
        grid=(_GSTEPS,),
        in_specs=in_specs,
        out_specs=pl.BlockSpec((1, _TPS, 128), lambda g, cref: (g, 0, 0)),
    )
    pblk = pl.pallas_call(
        _gather_body,
        grid_spec=grid_spec,
        out_shape=jax.ShapeDtypeStruct((_GSTEPS, _TPS, 128), jnp.float32),
    )(cblk, *([y_pred] * _TPS))

    out = pl.pallas_call(
        _loss_body,
        out_shape=jax.ShapeDtypeStruct((1, 1), jnp.float32),
    )(
        pblk.reshape(_NTOK, 128),
        yt.reshape(_NTOK, 1),
        um.reshape(1, _NTOK),
    )
    return out[0, 0]


# speedup vs baseline: 1.2747x; 1.2747x over previous
import jax
import jax.numpy as jnp
from jax import lax
from jax.experimental import pallas as pl
from jax.experimental.pallas import tpu as pltpu

_EPS = 1e-07
_VOCAB = 30522
_NTOK = 1024
_TPS = 128
_GSTEPS = _NTOK // _TPS


def _gather_body(cblk_ref, *refs):
    in_refs = refs[:_TPS]
    out_ref = refs[_TPS]
    for j in range(_TPS):
        out_ref[0, pl.ds(j, 1), :] = in_refs[j][0, pl.ds(j % 8, 1), :]
    del cblk_ref


def _loss_body(pblk_ref, yt_ref, um_ref, out_ref):
    yt = yt_ref[...]
    um = um_ref[...]
    pblk = pblk_ref[...]
    lane = lax.broadcasted_iota(jnp.int32, (_NTOK, 128), 1)
    p = jnp.sum(
        jnp.where(lane == (yt & 127), pblk, 0.0), axis=1, keepdims=True
    )
    cnt = jnp.sum((yt == um).astype(jnp.float32), axis=1, keepdims=True)
    alpha = jnp.where(
        (yt >= 2) & (cnt > 0.0),
        lax.rsqrt(jnp.maximum(cnt, 1e-20)),
        0.0,
    )
    keep = (yt != 0).astype(jnp.float32)
    a = alpha * keep
    pc = jnp.clip(p, _EPS, 1.0 - _EPS)
    om = 1.0 - pc
    num = jnp.sum(a * om * om * (-jnp.log(pc)))
    den = jnp.sum(a)
    out_ref[...] = (num / den).reshape(1, 1)


def kernel(y_pred, y_true):
    yt = y_true[:, :, 0].reshape(-1)
    um = y_true[:, :, 1].reshape(-1)
    cblk = yt >> 7

    in_specs = [
        pl.BlockSpec(
            (1, 8, 128),
            lambda g, cref, j=j: (
                (g * _TPS) // 128,
                ((g * _TPS) % 128) // 8 + j // 8,
                cref[_TPS * g + j],
            ),
        )
        for j in range(_TPS)
    ]
    grid_spec = pltpu.PrefetchScalarGridSpec(
        num_scalar_prefetch=1,
        grid=(_GSTEPS,),
        in_specs=in_specs,
        out_specs=pl.BlockSpec((1, _TPS, 128), lambda g, cref: (g, 0, 0)),
    )
    pblk = pl.pallas_call(
        _gather_body,
        grid_spec=grid_spec,
        out_shape=jax.ShapeDtypeStruct((_GSTEPS, _TPS, 128), jnp.float32),
    )(cblk, *([y_pred] * _TPS))

    out = pl.pallas_call(
        _loss_body,
        out_shape=jax.ShapeDtypeStruct((1, 1), jnp.float32),
    )(
        pblk.reshape(_NTOK, 128),
        yt.reshape(_NTOK, 1),
        um.reshape(1, _NTOK),
    )
    return out[0, 0]


# speedup vs baseline: 1.4580x; 1.1438x over previous
import jax
import jax.numpy as jnp
from jax import lax
from jax.experimental import pallas as pl
from jax.experimental.pallas import tpu as pltpu

_EPS = 1e-07
_VOCAB = 30522
_NTOK = 1024
_NSEM = 16


def _gather_body(cstart_ref, yp_ref, out_ref, scratch, sems):
    copies = []
    for t in range(_NTOK):
        b = t // 128
        r8 = ((t % 128) // 8) * 8
        copies.append(
            pltpu.make_async_copy(
                yp_ref.at[b].at[
                    pl.ds(r8, 8), pl.ds(pl.multiple_of(cstart_ref[t], 128), 128)
                ],
                scratch.at[t],
                sems.at[t % _NSEM],
            )
        )
        copies[-1].start()
    for cp in copies:
        cp.wait()
    x = scratch[...]
    sub = lax.broadcasted_iota(jnp.int32, (_NTOK, 8, 128), 1)
    row = lax.broadcasted_iota(jnp.int32, (_NTOK, 8, 128), 0) & 7
    out_ref[...] = jnp.sum(jnp.where(sub == row, x, 0.0), axis=1)


def _loss_body(pblk_ref, yt_ref, um_ref, out_ref):
    yt = yt_ref[...]
    um = um_ref[...]
    pblk = pblk_ref[...]
    lane = lax.broadcasted_iota(jnp.int32, (_NTOK, 128), 1)
    p = jnp.sum(
        jnp.where(lane == (yt & 127), pblk, 0.0), axis=1, keepdims=True
    )
    cnt = jnp.sum((yt == um).astype(jnp.float32), axis=1, keepdims=True)
    alpha = jnp.where(
        (yt >= 2) & (cnt > 0.0),
        lax.rsqrt(jnp.maximum(cnt, 1e-20)),
        0.0,
    )
    keep = (yt != 0).astype(jnp.float32)
    a = alpha * keep
    pc = jnp.clip(p, _EPS, 1.0 - _EPS)
    om = 1.0 - pc
    num = jnp.sum(a * om * om * (-jnp.log(pc)))
    den = jnp.sum(a)
    out_ref[...] = (num / den).reshape(1, 1)


def kernel(y_pred, y_true):
    yt = y_true[:, :, 0].reshape(-1)
    um = y_true[:, :, 1].reshape(-1)
    cstart = (yt >> 7) << 7

    grid_spec = pltpu.PrefetchScalarGridSpec(
        num_scalar_prefetch=1,
        grid=(1,),
        in_specs=[pl.BlockSpec(memory_space=pl.ANY)],
        out_specs=pl.BlockSpec((_NTOK, 128), lambda g, cref: (0, 0)),
        scratch_shapes=[
            pltpu.VMEM((_NTOK, 8, 128), jnp.float32),
            pltpu.SemaphoreType.DMA((_NSEM,)),
        ],
    )
    pblk = pl.pallas_call(
        _gather_body,
        grid_spec=grid_spec,
        out_shape=jax.ShapeDtypeStruct((_NTOK, 128), jnp.float32),
    )(cstart, y_pred)

    out = pl.pallas_call(
        _loss_body,
        out_shape=jax.ShapeDtypeStruct((1, 1), jnp.float32),
    )(
        pblk,
        yt.reshape(_NTOK, 1),
        um.reshape(1, _NTOK),
    )
    return out[0, 0]
